# baseline (speedup 1.0000x reference)
import functools

import jax
import jax.numpy as jnp
from jax.experimental import pallas as pl
from jax.experimental.pallas import tpu as pltpu

_B, _S, _D = 1, 2048, 1024
_E, _K, _F, _R = 8, 2, 512, 8
_T = _B * _S
_TC = 128
_EF = _E * _F


def _moe_body(x16_ref, wr_ref, wg_ref, wu_ref, wd_ref, ag_ref, bg_ref,
              au_ref, bu_ref, ad_ref, bd_ref, out_ref,
              comb_ref, h_ref, wdall_ref, wgu_ref):
    s = pl.program_id(0)

    @pl.when(s == 0)
    def _router():
        logits = jax.lax.dot_general(
            x16_ref[...], wr_ref[...].astype(jnp.bfloat16),
            (((1,), (1,)), ((), ())),
            preferred_element_type=jnp.float32)
        m = jnp.max(logits, axis=-1, keepdims=True)
        p = jnp.exp(logits - m)
        p = p / jnp.sum(p, axis=-1, keepdims=True)
        lane = jax.lax.broadcasted_iota(jnp.int32, (_T, _E), 1)
        p1 = jnp.max(p, axis=-1, keepdims=True)
        i1 = jnp.min(jnp.where(p == p1, lane, _E), axis=-1, keepdims=True)
        m1 = lane == i1
        pr = jnp.where(m1, -1.0, p)
        p2 = jnp.max(pr, axis=-1, keepdims=True)
        i2 = jnp.min(jnp.where(pr == p2, lane, _E), axis=-1, keepdims=True)
        m2 = lane == i2
        comb_ref[...] = ((jnp.where(m1, p, 0.0) + jnp.where(m2, p, 0.0))
                         / (p1 + p2)).astype(jnp.bfloat16)

    @pl.when(s < _E)
    def _merge():
        def merged(w, bt_, a_):
            lo = jax.lax.dot_general(bt_, a_, (((0,), (0,)), ((), ())),
                                     preferred_element_type=jnp.float32)
            return (w + lo).astype(jnp.bfloat16)

        buf = jax.lax.rem(s, 2)
        wgu_ref[buf, 0:_F, :] = merged(wg_ref[0], bg_ref[0], ag_ref[0])
        wgu_ref[buf, _F:2 * _F, :] = merged(wu_ref[0], bu_ref[0], au_ref[0])
        col = pl.multiple_of(s * _F, _F)
        wdall_ref[:, pl.ds(col, _F)] = merged(wd_ref[0], bd_ref[0], ad_ref[0])

    @pl.when((s >= 1) & (s <= _E))
    def _expert():
        e = s - 1
        buf = jax.lax.rem(e, 2)
        colh = pl.multiple_of(e * _F, _F)
        for c in range(_T // _TC):
            sl = pl.ds(c * _TC, _TC)
            gu = jax.lax.dot_general(x16_ref[sl, :], wgu_ref[buf],
                                     (((1,), (1,)), ((), ())),
                                     preferred_element_type=jnp.float32)
            g = gu[:, :_F]
            u = gu[:, _F:]
            lane = jax.lax.broadcasted_iota(jnp.int32, (_TC, _E), 1)
            cw = jnp.sum(jnp.where(lane == e, comb_ref[sl, :].astype(jnp.float32),
                                   0.0), axis=-1, keepdims=True)
            h_ref[sl, pl.ds(colh, _F)] = (
                g * (1.0 / (1.0 + jnp.exp(-g))) * u * cw).astype(jnp.bfloat16)

    @pl.when(s == _E + 1)
    def _down():
        for c in range(_T // _TC):
            sl = pl.ds(c * _TC, _TC)
            out_ref[sl, :] = jax.lax.dot_general(
                h_ref[sl, :], wdall_ref[...], (((1,), (1,)), ((), ())),
                preferred_element_type=jnp.float32)


@functools.partial(jax.jit, static_argnames=("interpret",))
def kernel(hidden_states, Wr, Wg, Wu, Wd, Ag, Bg, Au, Bu, Ad, Bd,
           interpret=False):
    x16 = hidden_states.reshape(_T, _D).astype(jnp.bfloat16)
    BgT = jnp.swapaxes(Bg, 1, 2)
    BuT = jnp.swapaxes(Bu, 1, 2)
    BdT = jnp.swapaxes(Bd, 1, 2)

    def eb(s):
        return jnp.minimum(s, _E - 1)

    y = pl.pallas_call(
        _moe_body,
        grid=(_E + 2,),
        in_specs=[
            pl.BlockSpec((_T, _D), lambda s: (0, 0)),
            pl.BlockSpec((_E, _D), lambda s: (0, 0)),
            pl.BlockSpec((1, _F, _D), lambda s: (eb(s), 0, 0)),
            pl.BlockSpec((1, _F, _D), lambda s: (eb(s), 0, 0)),
            pl.BlockSpec((1, _D, _F), lambda s: (eb(s), 0, 0)),
            pl.BlockSpec((1, _R, _D), lambda s: (eb(s), 0, 0)),
            pl.BlockSpec((1, _R, _F), lambda s: (eb(s), 0, 0)),
            pl.BlockSpec((1, _R, _D), lambda s: (eb(s), 0, 0)),
            pl.BlockSpec((1, _R, _F), lambda s: (eb(s), 0, 0)),
            pl.BlockSpec((1, _R, _F), lambda s: (eb(s), 0, 0)),
            pl.BlockSpec((1, _R, _D), lambda s: (eb(s), 0, 0)),
        ],
        out_specs=pl.BlockSpec((_T, _D), lambda s: (0, 0)),
        out_shape=jax.ShapeDtypeStruct((_T, _D), jnp.float32),
        scratch_shapes=[pltpu.VMEM((_T, _E), jnp.bfloat16),
                        pltpu.VMEM((_T, _EF), jnp.bfloat16),
                        pltpu.VMEM((_D, _EF), jnp.bfloat16),
                        pltpu.VMEM((2, 2 * _F, _D), jnp.bfloat16)],
        interpret=interpret,
    )(x16, Wr, Wg, Wu, Wd, Ag, BgT, Au, BuT, Ad, BdT)

    return y.reshape(_B, _S, _D)


# speedup vs baseline: 2.3986x; 2.3986x over previous
import functools

import jax
import jax.numpy as jnp
from jax.experimental import pallas as pl
from jax.experimental.pallas import tpu as pltpu

_B, _S, _D = 1, 2048, 1024
_E, _K, _F, _R = 8, 2, 512, 8
_T = _B * _S
_TC = 256
_EF = _E * _F


def _moe_body(x16_ref, wr_ref, wg_ref, wu_ref, wd_ref, ag_ref, bg_ref,
              au_ref, bu_ref, ad_ref, bd_ref, out_ref,
              comb_ref, h_ref, wdall_ref, wgu_ref):
    s = pl.program_id(0)

    @pl.when(s == 0)
    def _router():
        logits = jax.lax.dot_general(
            x16_ref[...], wr_ref[...].astype(jnp.bfloat16),
            (((1,), (1,)), ((), ())),
            preferred_element_type=jnp.float32)
        m = jnp.max(logits, axis=-1, keepdims=True)
        p = jnp.exp(logits - m)
        p = p / jnp.sum(p, axis=-1, keepdims=True)
        lane = jax.lax.broadcasted_iota(jnp.int32, (_T, _E), 1)
        p1 = jnp.max(p, axis=-1, keepdims=True)
        i1 = jnp.min(jnp.where(p == p1, lane, _E), axis=-1, keepdims=True)
        m1 = lane == i1
        pr = jnp.where(m1, -1.0, p)
        p2 = jnp.max(pr, axis=-1, keepdims=True)
        i2 = jnp.min(jnp.where(pr == p2, lane, _E), axis=-1, keepdims=True)
        m2 = lane == i2
        comb_ref[...] = ((jnp.where(m1, p, 0.0) + jnp.where(m2, p, 0.0))
                         / (p1 + p2)).astype(jnp.bfloat16)

    @pl.when(s < _E)
    def _merge():
        def merged(w, bt_, a_):
            lo = jax.lax.dot_general(bt_, a_, (((0,), (0,)), ((), ())),
                                     preferred_element_type=jnp.float32)
            return (w + lo).astype(jnp.bfloat16)

        buf = jax.lax.rem(s, 2)
        wgu_ref[buf, 0:_F, :] = merged(wg_ref[0], bg_ref[0], ag_ref[0])
        wgu_ref[buf, _F:2 * _F, :] = merged(wu_ref[0], bu_ref[0], au_ref[0])
        col = pl.multiple_of(s * _F, _F)
        wdall_ref[:, pl.ds(col, _F)] = merged(wd_ref[0], bd_ref[0], ad_ref[0])

    @pl.when((s >= 1) & (s <= _E))
    def _expert():
        e = s - 1
        buf = jax.lax.rem(e, 2)
        colh = pl.multiple_of(e * _F, _F)
        for c in range(_T // _TC):
            sl = pl.ds(c * _TC, _TC)
            gu = jax.lax.dot_general(x16_ref[sl, :], wgu_ref[buf],
                                     (((1,), (1,)), ((), ())),
                                     preferred_element_type=jnp.float32)
            g = gu[:, :_F]
            u = gu[:, _F:]
            lane = jax.lax.broadcasted_iota(jnp.int32, (_TC, _E), 1)
            cw = jnp.sum(jnp.where(lane == e, comb_ref[sl, :].astype(jnp.float32),
                                   0.0), axis=-1, keepdims=True)
            h_ref[sl, pl.ds(colh, _F)] = (
                g * jax.nn.sigmoid(g) * u * cw).astype(jnp.bfloat16)

    @pl.when(s == _E + 1)
    def _down():
        for c in range(_T // _TC):
            sl = pl.ds(c * _TC, _TC)
            out_ref[sl, :] = jax.lax.dot_general(
                h_ref[sl, :], wdall_ref[...], (((1,), (1,)), ((), ())),
                preferred_element_type=jnp.float32)


@functools.partial(jax.jit, static_argnames=("interpret",))
def kernel(hidden_states, Wr, Wg, Wu, Wd, Ag, Bg, Au, Bu, Ad, Bd,
           interpret=False):
    x16 = hidden_states.reshape(_T, _D).astype(jnp.bfloat16)
    BgT = jnp.swapaxes(Bg, 1, 2)
    BuT = jnp.swapaxes(Bu, 1, 2)
    BdT = jnp.swapaxes(Bd, 1, 2)

    def eb(s):
        return jnp.minimum(s, _E - 1)

    y = pl.pallas_call(
        _moe_body,
        grid=(_E + 2,),
        in_specs=[
            pl.BlockSpec((_T, _D), lambda s: (0, 0)),
            pl.BlockSpec((_E, _D), lambda s: (0, 0)),
            pl.BlockSpec((1, _F, _D), lambda s: (eb(s), 0, 0)),
            pl.BlockSpec((1, _F, _D), lambda s: (eb(s), 0, 0)),
            pl.BlockSpec((1, _D, _F), lambda s: (eb(s), 0, 0)),
            pl.BlockSpec((1, _R, _D), lambda s: (eb(s), 0, 0)),
            pl.BlockSpec((1, _R, _F), lambda s: (eb(s), 0, 0)),
            pl.BlockSpec((1, _R, _D), lambda s: (eb(s), 0, 0)),
            pl.BlockSpec((1, _R, _F), lambda s: (eb(s), 0, 0)),
            pl.BlockSpec((1, _R, _F), lambda s: (eb(s), 0, 0)),
            pl.BlockSpec((1, _R, _D), lambda s: (eb(s), 0, 0)),
        ],
        out_specs=pl.BlockSpec((_T, _D), lambda s: (0, 0)),
        out_shape=jax.ShapeDtypeStruct((_T, _D), jnp.float32),
        scratch_shapes=[pltpu.VMEM((_T, _E), jnp.bfloat16),
                        pltpu.VMEM((_T, _EF), jnp.bfloat16),
                        pltpu.VMEM((_D, _EF), jnp.bfloat16),
                        pltpu.VMEM((2, 2 * _F, _D), jnp.bfloat16)],
        interpret=interpret,
    )(x16, Wr, Wg, Wu, Wd, Ag, BgT, Au, BuT, Ad, BdT)

    return y.reshape(_B, _S, _D)
